# baseline (speedup 1.0000x reference)
import functools

import jax
import jax.numpy as jnp
from jax import lax
from jax.experimental import pallas as pl
from jax.experimental.pallas import tpu as pltpu
from jax.experimental.pallas import tpu_sc as plsc

F = 40960
B = 1024
BK = 2048
NK = F // BK


NW = 32
CH = 8192
NCHUNK = F // CH
GP = 256
GROUPS = CH // GP
NHMAX = NCHUNK * GROUPS
KPAD = F + 16
D = 256
GCH = 16


def _sc_l1_body(xr_hbm, w1t_hbm, out_hbm, xbuf, idxbuf, pkbuf, lmbuf, gbuf,
                rows, acc, row0, semx0, semx1, semg):
    nsc_rows = out_hbm.shape[0]
    rpw = nsc_rows // NW
    row_base = xr_hbm.shape[0] - nsc_rows
    wid = lax.axis_index("s") * 2 + lax.axis_index("c")

    pltpu.sync_copy(w1t_hbm.at[0], row0)

    zf = jnp.zeros((16,), jnp.float32)
    zi = jnp.zeros((16,), jnp.int32)
    iota = lax.iota(jnp.int32, 16)

    def hsum(v):
        for sh in (8, 4, 2, 1):
            v = v + jnp.take(v, iota ^ sh)
        return v

    def _ctz(b):
        bf = b.astype(jnp.float32)
        return (lax.bitcast_convert_type(bf, jnp.int32) >> 23) - 127

    def do_row(i, _):
        lrow = wid * rpw + i
        r = row_base + lrow

        cps = [None, None]
        cps[0] = pltpu.async_copy(xr_hbm.at[r, pl.ds(0, CH)], xbuf.at[0],
                                  semx0)
        hn = (jnp.int32(0), jnp.int32(0))
        for c in range(NCHUNK):
            if c + 1 < NCHUNK:
                sem = semx1 if (c + 1) % 2 else semx0
                cps[(c + 1) % 2] = pltpu.async_copy(
                    xr_hbm.at[r, pl.ds((c + 1) * CH, CH)],
                    xbuf.at[(c + 1) % 2], sem)
            cps[c % 2].wait()
            cbase = c * CH

            def scan_group(g, hn, _c=c, _cbase=cbase):
                base = g * GP
                vs = [xbuf[_c % 2, pl.ds(base + 16 * t, 16)]
                      for t in range(16)]
                tot = vs[0]
                for v in vs[1:]:
                    tot = tot + v
                s = hsum(tot)[0]

                def hit(hn):
                    h, nb = hn
                    pk = vs[0]
                    for t in range(1, 16):
                        pk = pk + vs[t] * float(1 << t)
                    p2 = jnp.ones((16,), jnp.float32)
                    for sh in (1, 2, 4, 8):
                        p2 = p2 * jnp.where((iota & sh) != 0, 2.0 ** sh,
                                            1.0)
                    lmf = hsum(jnp.where(pk > 0.0, p2, 0.0))[0]
                    pkbuf[pl.ds(h * 16, 16)] = pk
                    lmbuf[pl.ds(h, 16)] = jnp.full((16,),
                                                   lmf.astype(jnp.int32),
                                                   jnp.int32)
                    gbuf[pl.ds(h, 16)] = jnp.full((16,), _cbase + base,
                                                  jnp.int32)
                    return (h + 1, nb + s.astype(jnp.int32))

                return lax.cond(s > 0.0, hit, lambda hn: hn, hn)

            hn = lax.fori_loop(0, GROUPS, scan_group, hn)
        h, nb = hn

        def pop_body(it, carry):
            hh, j, lm, w, n2 = carry
            adv_g = (w == 0) & (lm == 0)
            hh = jnp.where(adv_g, hh + 1, hh)
            lm = jnp.where(adv_g, lmbuf[pl.ds(hh, 16)][0], lm)
            gbase = gbuf[pl.ds(hh, 16)][0]
            adv_l = w == 0
            jn = _ctz(lm & (-lm))
            j = jnp.where(adv_l, jn, j)
            lm = jnp.where(adv_l, lm & (lm - 1), lm)
            wv = pkbuf[pl.ds(hh * 16 + j, 16)][0]
            w = jnp.where(adv_l, wv.astype(jnp.int32), w)
            bp = _ctz(w & (-w))
            col = gbase + 16 * bp + j
            idxbuf[pl.ds(n2, 16)] = jnp.full((16,), col, jnp.int32)
            return (hh, j, lm, w & (w - 1), n2 + 1)

        carry = (jnp.int32(-1), jnp.int32(0), jnp.int32(0), jnp.int32(0),
                 jnp.int32(0))
        carry = lax.fori_loop(0, nb, pop_body, carry)
        n = carry[4]

        idxbuf[pl.ds(n, 16)] = zi
        for t in range(16):
            acc[pl.ds(16 * t, 16)] = zf
        nch = (n + 15) >> 4

        def gbody(j, _):
            pltpu.async_copy(w1t_hbm.at[idxbuf.at[pl.ds(j * GCH, GCH)]],
                             rows, semg).wait()
            for jr in range(GCH):
                for t in range(16):
                    plsc.addupdate(acc.at[pl.ds(16 * t, 16)],
                                   rows[jr, pl.ds(16 * t, 16)])
            return 0

        lax.fori_loop(0, nch, gbody, 0)

        padf = (nch * GCH - n).astype(jnp.float32)
        pv = jnp.full((16,), padf, jnp.float32)
        for t in range(16):
            sl = pl.ds(16 * t, 16)
            acc[sl] = acc[sl] - pv * row0[sl]

        pltpu.sync_copy(acc, out_hbm.at[lrow])
        return 0

    lax.fori_loop(0, rpw, do_row, 0)


def _sc_l1(xr, w1t, nsc_rows):
    mesh = plsc.VectorSubcoreMesh(core_axis_name="c", subcore_axis_name="s")
    kfn = functools.partial(
        pl.kernel, mesh=mesh,
        out_type=jax.ShapeDtypeStruct((nsc_rows, D), jnp.float32),
        scratch_types=[
            pltpu.VMEM((2, CH), jnp.float32),
            pltpu.VMEM((KPAD,), jnp.int32),
            pltpu.VMEM((NHMAX * 16 + 16,), jnp.float32),
            pltpu.VMEM((NHMAX + 16,), jnp.int32),
            pltpu.VMEM((NHMAX + 16,), jnp.int32),
            pltpu.VMEM((GCH, D), jnp.float32),
            pltpu.VMEM((D,), jnp.float32),
            pltpu.VMEM((D,), jnp.float32),
            pltpu.SemaphoreType.DMA,
            pltpu.SemaphoreType.DMA,
            pltpu.SemaphoreType.DMA,
        ],
    )(_sc_l1_body)
    return kfn(xr, w1t)


def _l1_body(x1_ref, x2_ref, w1_ref, h1_ref, h2_ref):
    k = pl.program_id(0)
    wb = w1_ref[...].astype(jnp.bfloat16)
    x1b = x1_ref[...].astype(jnp.bfloat16)
    x2b = x2_ref[...].astype(jnp.bfloat16)
    dn = (((1,), (1,)), ((), ()))
    a1 = lax.dot_general(x1b, wb, dn, preferred_element_type=jnp.float32)
    a2 = lax.dot_general(x2b, wb, dn, preferred_element_type=jnp.float32)

    @pl.when(k == 0)
    def _():
        h1_ref[...] = a1
        h2_ref[...] = a2

    @pl.when(k > 0)
    def _():
        h1_ref[...] += a1
        h2_ref[...] += a2


def _ln_lrelu(v):
    mu = jnp.mean(v, axis=1, keepdims=True)
    var = jnp.mean((v - mu) ** 2, axis=1, keepdims=True)
    y = (v - mu) * lax.rsqrt(var)
    return jnp.maximum(0.05 * y, y)


def _mlp_body(h_ref, w2_ref, w3_ref, w4_ref, out_ref):
    dn = (((1,), (1,)), ((), ()))
    g1 = _ln_lrelu(h_ref[:, :256])
    g2 = _ln_lrelu(h_ref[:, 256:])
    w2 = w2_ref[...]
    m1 = _ln_lrelu(lax.dot_general(g1, w2, dn, preferred_element_type=jnp.float32))
    m2 = _ln_lrelu(lax.dot_general(g2, w2, dn, preferred_element_type=jnp.float32))
    w3a = w3_ref[:, :64]
    w3b = w3_ref[:, 64:]
    s = (lax.dot_general(m1, w3a, dn, preferred_element_type=jnp.float32)
         + lax.dot_general(m2, w3b, dn, preferred_element_type=jnp.float32))
    s = _ln_lrelu(s)
    out_ref[...] = lax.dot_general(s, w4_ref[...], dn,
                                   preferred_element_type=jnp.float32)


TC_B = 0


@jax.jit
def kernel(x, W1, W2, W3, W4):
    w1t = W1.T
    xr = x.reshape(2 * B, F)

    if TC_B > 0:
        h1_tc, h2_tc = pl.pallas_call(
            _l1_body,
            grid=(NK,),
            in_specs=[
                pl.BlockSpec((TC_B, BK), lambda k: (0, k)),
                pl.BlockSpec((TC_B, BK), lambda k: (0, NK + k)),
                pl.BlockSpec((256, BK), lambda k: (0, k)),
            ],
            out_specs=[
                pl.BlockSpec((TC_B, 256), lambda k: (0, 0)),
                pl.BlockSpec((TC_B, 256), lambda k: (0, 0)),
            ],
            out_shape=[
                jax.ShapeDtypeStruct((TC_B, 256), jnp.float32),
                jax.ShapeDtypeStruct((TC_B, 256), jnp.float32),
            ],
        )(x, x, W1)
        hcat_tc = jnp.concatenate([h1_tc, h2_tc], axis=1)

    nsc_rows = 2 * (B - TC_B)
    if nsc_rows > 0:
        h_sc = _sc_l1(xr, w1t, nsc_rows)
        hcat_sc = h_sc.reshape(B - TC_B, 512)

    if TC_B == 0:
        hcat = hcat_sc
    elif nsc_rows == 0:
        hcat = hcat_tc
    else:
        hcat = jnp.concatenate([hcat_tc, hcat_sc], axis=0)

    out = pl.pallas_call(
        _mlp_body,
        out_shape=jax.ShapeDtypeStruct((B, 1), jnp.float32),
    )(hcat, W2, W3, W4)
    return out


# speedup vs baseline: 4.0591x; 4.0591x over previous
import functools

import jax
import jax.numpy as jnp
from jax import lax
from jax.experimental import pallas as pl
from jax.experimental.pallas import tpu as pltpu
from jax.experimental.pallas import tpu_sc as plsc

F = 40960
B = 1024
BK = 2048
NK = F // BK


NW = 32
CH = 8192
NCHUNK = F // CH
GP = 256
GROUPS = CH // GP
NHMAX = NCHUNK * GROUPS
KPAD = F + 16
D = 256
GCH = 16


def _sc_l1_body(xr_hbm, w1t_hbm, out_hbm, xbuf, idxbuf, pkbuf, lmbuf, gbuf,
                rows, acc, row0, semx0, semx1, semg):
    nsc_rows = out_hbm.shape[0]
    rpw = nsc_rows // NW
    row_base = xr_hbm.shape[0] - nsc_rows
    wid = lax.axis_index("s") * 2 + lax.axis_index("c")

    pltpu.sync_copy(w1t_hbm.at[0], row0)

    zf = jnp.zeros((16,), jnp.float32)
    zi = jnp.zeros((16,), jnp.int32)
    iota = lax.iota(jnp.int32, 16)

    def hsum(v):
        for sh in (8, 4, 2, 1):
            v = v + jnp.take(v, iota ^ sh)
        return v

    def _ctz(b):
        bf = b.astype(jnp.float32)
        return (lax.bitcast_convert_type(bf, jnp.int32) >> 23) - 127

    def do_row(i, _):
        lrow = wid * rpw + i
        r = row_base + lrow

        cps = [None, None]
        cps[0] = pltpu.async_copy(xr_hbm.at[r, pl.ds(0, CH)], xbuf.at[0],
                                  semx0)
        hn = (jnp.int32(0), jnp.int32(0))
        for c in range(NCHUNK):
            if c + 1 < NCHUNK:
                sem = semx1 if (c + 1) % 2 else semx0
                cps[(c + 1) % 2] = pltpu.async_copy(
                    xr_hbm.at[r, pl.ds((c + 1) * CH, CH)],
                    xbuf.at[(c + 1) % 2], sem)
            cps[c % 2].wait()
            cbase = c * CH

            def scan_group(g, hn, _c=c, _cbase=cbase):
                base = g * GP
                vs = [xbuf[_c % 2, pl.ds(base + 16 * t, 16)]
                      for t in range(16)]
                tot = vs[0]
                for v in vs[1:]:
                    tot = tot + v
                s = hsum(tot)[0]

                def hit(hn):
                    h, nb = hn
                    pk = vs[0]
                    for t in range(1, 16):
                        pk = pk + vs[t] * float(1 << t)
                    p2 = jnp.ones((16,), jnp.float32)
                    for sh in (1, 2, 4, 8):
                        p2 = p2 * jnp.where((iota & sh) != 0, 2.0 ** sh,
                                            1.0)
                    lmf = hsum(jnp.where(pk > 0.0, p2, 0.0))[0]
                    pkbuf[pl.ds(h * 16, 16)] = pk
                    lmbuf[pl.ds(h, 16)] = jnp.full((16,),
                                                   lmf.astype(jnp.int32),
                                                   jnp.int32)
                    gbuf[pl.ds(h, 16)] = jnp.full((16,), _cbase + base,
                                                  jnp.int32)
                    return (h + 1, nb + s.astype(jnp.int32))

                return lax.cond(s > 0.0, hit, lambda hn: hn, hn)

            hn = lax.fori_loop(0, GROUPS, scan_group, hn)
        h, nb = hn

        def pop_body(it, carry):
            hh, j, lm, w, n2 = carry
            adv_g = (w == 0) & (lm == 0)
            hh = jnp.where(adv_g, hh + 1, hh)
            lm = jnp.where(adv_g, lmbuf[pl.ds(hh, 16)][0], lm)
            gbase = gbuf[pl.ds(hh, 16)][0]
            adv_l = w == 0
            jn = _ctz(lm & (-lm))
            j = jnp.where(adv_l, jn, j)
            lm = jnp.where(adv_l, lm & (lm - 1), lm)
            wv = pkbuf[pl.ds(hh * 16 + j, 16)][0]
            w = jnp.where(adv_l, wv.astype(jnp.int32), w)
            bp = _ctz(w & (-w))
            col = gbase + 16 * bp + j
            idxbuf[pl.ds(n2, 16)] = jnp.full((16,), col, jnp.int32)
            return (hh, j, lm, w & (w - 1), n2 + 1)

        carry = (jnp.int32(-1), jnp.int32(0), jnp.int32(0), jnp.int32(0),
                 jnp.int32(0))
        carry = lax.fori_loop(0, nb, pop_body, carry)
        n = carry[4]

        idxbuf[pl.ds(n, 16)] = zi
        for t in range(16):
            acc[pl.ds(16 * t, 16)] = zf
        nch = (n + 15) >> 4

        def gbody(j, _):
            pltpu.async_copy(w1t_hbm.at[idxbuf.at[pl.ds(j * GCH, GCH)]],
                             rows, semg).wait()
            for jr in range(GCH):
                for t in range(16):
                    plsc.addupdate(acc.at[pl.ds(16 * t, 16)],
                                   rows[jr, pl.ds(16 * t, 16)])
            return 0

        lax.fori_loop(0, nch, gbody, 0)

        padf = (nch * GCH - n).astype(jnp.float32)
        pv = jnp.full((16,), padf, jnp.float32)
        for t in range(16):
            sl = pl.ds(16 * t, 16)
            acc[sl] = acc[sl] - pv * row0[sl]

        pltpu.sync_copy(acc, out_hbm.at[lrow])
        return 0

    lax.fori_loop(0, rpw, do_row, 0)


def _sc_l1(xr, w1t, nsc_rows):
    mesh = plsc.VectorSubcoreMesh(core_axis_name="c", subcore_axis_name="s")
    kfn = functools.partial(
        pl.kernel, mesh=mesh,
        out_type=jax.ShapeDtypeStruct((nsc_rows, D), jnp.float32),
        scratch_types=[
            pltpu.VMEM((2, CH), jnp.float32),
            pltpu.VMEM((KPAD,), jnp.int32),
            pltpu.VMEM((NHMAX * 16 + 16,), jnp.float32),
            pltpu.VMEM((NHMAX + 16,), jnp.int32),
            pltpu.VMEM((NHMAX + 16,), jnp.int32),
            pltpu.VMEM((GCH, D), jnp.float32),
            pltpu.VMEM((D,), jnp.float32),
            pltpu.VMEM((D,), jnp.float32),
            pltpu.SemaphoreType.DMA,
            pltpu.SemaphoreType.DMA,
            pltpu.SemaphoreType.DMA,
        ],
    )(_sc_l1_body)
    return kfn(xr, w1t)


def _l1_body(x1_ref, x2_ref, w1_ref, h1_ref, h2_ref):
    k = pl.program_id(0)
    wb = w1_ref[...].astype(jnp.bfloat16)
    x1b = x1_ref[...].astype(jnp.bfloat16)
    x2b = x2_ref[...].astype(jnp.bfloat16)
    dn = (((1,), (1,)), ((), ()))
    a1 = lax.dot_general(x1b, wb, dn, preferred_element_type=jnp.float32)
    a2 = lax.dot_general(x2b, wb, dn, preferred_element_type=jnp.float32)

    @pl.when(k == 0)
    def _():
        h1_ref[...] = a1
        h2_ref[...] = a2

    @pl.when(k > 0)
    def _():
        h1_ref[...] += a1
        h2_ref[...] += a2


def _ln_lrelu(v):
    mu = jnp.mean(v, axis=1, keepdims=True)
    var = jnp.mean((v - mu) ** 2, axis=1, keepdims=True)
    y = (v - mu) * lax.rsqrt(var)
    return jnp.maximum(0.05 * y, y)


def _mlp_body(h_ref, w2_ref, w3_ref, w4_ref, out_ref):
    dn = (((1,), (1,)), ((), ()))
    g1 = _ln_lrelu(h_ref[:, :256])
    g2 = _ln_lrelu(h_ref[:, 256:])
    w2 = w2_ref[...]
    m1 = _ln_lrelu(lax.dot_general(g1, w2, dn, preferred_element_type=jnp.float32))
    m2 = _ln_lrelu(lax.dot_general(g2, w2, dn, preferred_element_type=jnp.float32))
    w3a = w3_ref[:, :64]
    w3b = w3_ref[:, 64:]
    s = (lax.dot_general(m1, w3a, dn, preferred_element_type=jnp.float32)
         + lax.dot_general(m2, w3b, dn, preferred_element_type=jnp.float32))
    s = _ln_lrelu(s)
    out_ref[...] = lax.dot_general(s, w4_ref[...], dn,
                                   preferred_element_type=jnp.float32)


TC_B = 960


@jax.jit
def kernel(x, W1, W2, W3, W4):
    w1t = W1.T
    xr = x.reshape(2 * B, F)

    if TC_B > 0:
        h1_tc, h2_tc = pl.pallas_call(
            _l1_body,
            grid=(NK,),
            in_specs=[
                pl.BlockSpec((TC_B, BK), lambda k: (0, k)),
                pl.BlockSpec((TC_B, BK), lambda k: (0, NK + k)),
                pl.BlockSpec((256, BK), lambda k: (0, k)),
            ],
            out_specs=[
                pl.BlockSpec((TC_B, 256), lambda k: (0, 0)),
                pl.BlockSpec((TC_B, 256), lambda k: (0, 0)),
            ],
            out_shape=[
                jax.ShapeDtypeStruct((TC_B, 256), jnp.float32),
                jax.ShapeDtypeStruct((TC_B, 256), jnp.float32),
            ],
        )(x, x, W1)
        hcat_tc = jnp.concatenate([h1_tc, h2_tc], axis=1)

    nsc_rows = 2 * (B - TC_B)
    if nsc_rows > 0:
        h_sc = _sc_l1(xr, w1t, nsc_rows)
        hcat_sc = h_sc.reshape(B - TC_B, 512)

    if TC_B == 0:
        hcat = hcat_sc
    elif nsc_rows == 0:
        hcat = hcat_tc
    else:
        hcat = jnp.concatenate([hcat_tc, hcat_sc], axis=0)

    out = pl.pallas_call(
        _mlp_body,
        out_shape=jax.ShapeDtypeStruct((B, 1), jnp.float32),
    )(hcat, W2, W3, W4)
    return out


# speedup vs baseline: 4.2231x; 1.0404x over previous
import functools

import jax
import jax.numpy as jnp
from jax import lax
from jax.experimental import pallas as pl
from jax.experimental.pallas import tpu as pltpu
from jax.experimental.pallas import tpu_sc as plsc

F = 40960
B = 1024
BK = 2048
NK = F // BK


NW = 32
CH = 8192
NCHUNK = F // CH
GP = 256
GROUPS = CH // GP
NHMAX = NCHUNK * GROUPS
KPAD = F + 16
D = 256
GCH = 16


def _sc_l1_body(xr_hbm, w1t_hbm, out_hbm, xbuf, idxbuf, pkbuf, lmbuf, gbuf,
                rows, acc, row0, semx0, semx1, semg):
    nsc_rows = out_hbm.shape[0]
    rpw = nsc_rows // NW
    row_base = xr_hbm.shape[0] - nsc_rows
    wid = lax.axis_index("s") * 2 + lax.axis_index("c")

    pltpu.sync_copy(w1t_hbm.at[0], row0)

    zf = jnp.zeros((16,), jnp.float32)
    zi = jnp.zeros((16,), jnp.int32)
    iota = lax.iota(jnp.int32, 16)

    def hsum(v):
        for sh in (8, 4, 2, 1):
            v = v + jnp.take(v, iota ^ sh)
        return v

    def _ctz(b):
        bf = b.astype(jnp.float32)
        return (lax.bitcast_convert_type(bf, jnp.int32) >> 23) - 127

    def do_row(i, _):
        lrow = wid * rpw + i
        r = row_base + lrow

        cps = [None, None]
        cps[0] = pltpu.async_copy(xr_hbm.at[r, pl.ds(0, CH)], xbuf.at[0],
                                  semx0)
        hn = (jnp.int32(0), jnp.int32(0))
        for c in range(NCHUNK):
            if c + 1 < NCHUNK:
                sem = semx1 if (c + 1) % 2 else semx0
                cps[(c + 1) % 2] = pltpu.async_copy(
                    xr_hbm.at[r, pl.ds((c + 1) * CH, CH)],
                    xbuf.at[(c + 1) % 2], sem)
            cps[c % 2].wait()
            cbase = c * CH

            def scan_group(g, hn, _c=c, _cbase=cbase):
                base = g * GP
                vs = [xbuf[_c % 2, pl.ds(base + 16 * t, 16)]
                      for t in range(16)]
                tot = vs[0]
                for v in vs[1:]:
                    tot = tot + v
                s = hsum(tot)[0]

                def hit(hn):
                    h, nb = hn
                    pk = vs[0]
                    for t in range(1, 16):
                        pk = pk + vs[t] * float(1 << t)
                    p2 = jnp.ones((16,), jnp.float32)
                    for sh in (1, 2, 4, 8):
                        p2 = p2 * jnp.where((iota & sh) != 0, 2.0 ** sh,
                                            1.0)
                    lmf = hsum(jnp.where(pk > 0.0, p2, 0.0))[0]
                    pkbuf[pl.ds(h * 16, 16)] = pk
                    lmbuf[pl.ds(h, 16)] = jnp.full((16,),
                                                   lmf.astype(jnp.int32),
                                                   jnp.int32)
                    gbuf[pl.ds(h, 16)] = jnp.full((16,), _cbase + base,
                                                  jnp.int32)
                    return (h + 1, nb + s.astype(jnp.int32))

                return lax.cond(s > 0.0, hit, lambda hn: hn, hn)

            hn = lax.fori_loop(0, GROUPS, scan_group, hn)
        h, nb = hn

        def pop_body(it, carry):
            hh, j, lm, w, n2 = carry
            adv_g = (w == 0) & (lm == 0)
            hh = jnp.where(adv_g, hh + 1, hh)
            lm = jnp.where(adv_g, lmbuf[pl.ds(hh, 16)][0], lm)
            gbase = gbuf[pl.ds(hh, 16)][0]
            adv_l = w == 0
            jn = _ctz(lm & (-lm))
            j = jnp.where(adv_l, jn, j)
            lm = jnp.where(adv_l, lm & (lm - 1), lm)
            wv = pkbuf[pl.ds(hh * 16 + j, 16)][0]
            w = jnp.where(adv_l, wv.astype(jnp.int32), w)
            bp = _ctz(w & (-w))
            col = gbase + 16 * bp + j
            idxbuf[pl.ds(n2, 16)] = jnp.full((16,), col, jnp.int32)
            return (hh, j, lm, w & (w - 1), n2 + 1)

        carry = (jnp.int32(-1), jnp.int32(0), jnp.int32(0), jnp.int32(0),
                 jnp.int32(0))
        carry = lax.fori_loop(0, nb, pop_body, carry)
        n = carry[4]

        idxbuf[pl.ds(n, 16)] = zi
        for t in range(16):
            acc[pl.ds(16 * t, 16)] = zf
        nch = (n + 15) >> 4

        def gbody(j, _):
            pltpu.async_copy(w1t_hbm.at[idxbuf.at[pl.ds(j * GCH, GCH)]],
                             rows, semg).wait()
            for jr in range(GCH):
                for t in range(16):
                    plsc.addupdate(acc.at[pl.ds(16 * t, 16)],
                                   rows[jr, pl.ds(16 * t, 16)])
            return 0

        lax.fori_loop(0, nch, gbody, 0)

        padf = (nch * GCH - n).astype(jnp.float32)
        pv = jnp.full((16,), padf, jnp.float32)
        for t in range(16):
            sl = pl.ds(16 * t, 16)
            acc[sl] = acc[sl] - pv * row0[sl]

        pltpu.sync_copy(acc, out_hbm.at[lrow])
        return 0

    lax.fori_loop(0, rpw, do_row, 0)


def _sc_l1(xr, w1t, nsc_rows):
    mesh = plsc.VectorSubcoreMesh(core_axis_name="c", subcore_axis_name="s")
    kfn = functools.partial(
        pl.kernel, mesh=mesh,
        out_type=jax.ShapeDtypeStruct((nsc_rows, D), jnp.float32),
        scratch_types=[
            pltpu.VMEM((2, CH), jnp.float32),
            pltpu.VMEM((KPAD,), jnp.int32),
            pltpu.VMEM((NHMAX * 16 + 16,), jnp.float32),
            pltpu.VMEM((NHMAX + 16,), jnp.int32),
            pltpu.VMEM((NHMAX + 16,), jnp.int32),
            pltpu.VMEM((GCH, D), jnp.float32),
            pltpu.VMEM((D,), jnp.float32),
            pltpu.VMEM((D,), jnp.float32),
            pltpu.SemaphoreType.DMA,
            pltpu.SemaphoreType.DMA,
            pltpu.SemaphoreType.DMA,
        ],
    )(_sc_l1_body)
    return kfn(xr, w1t)


def _tr_body(w_ref, o_ref):
    o_ref[...] = jnp.swapaxes(w_ref[...], 0, 1)


def _w1t(W1):
    return pl.pallas_call(
        _tr_body,
        grid=(NK,),
        in_specs=[pl.BlockSpec((256, BK), lambda k: (0, k))],
        out_specs=pl.BlockSpec((BK, 256), lambda k: (k, 0)),
        out_shape=jax.ShapeDtypeStruct((F, 256), jnp.float32),
    )(W1)


def _l1_body(x1_ref, x2_ref, w1_ref, h1_ref, h2_ref):
    k = pl.program_id(0)
    wb = w1_ref[...].astype(jnp.bfloat16)
    x1b = x1_ref[...].astype(jnp.bfloat16)
    x2b = x2_ref[...].astype(jnp.bfloat16)
    dn = (((1,), (1,)), ((), ()))
    a1 = lax.dot_general(x1b, wb, dn, preferred_element_type=jnp.float32)
    a2 = lax.dot_general(x2b, wb, dn, preferred_element_type=jnp.float32)

    @pl.when(k == 0)
    def _():
        h1_ref[...] = a1
        h2_ref[...] = a2

    @pl.when(k > 0)
    def _():
        h1_ref[...] += a1
        h2_ref[...] += a2


def _ln_lrelu(v):
    mu = jnp.mean(v, axis=1, keepdims=True)
    var = jnp.mean((v - mu) ** 2, axis=1, keepdims=True)
    y = (v - mu) * lax.rsqrt(var)
    return jnp.maximum(0.05 * y, y)


def _mlp_body(h_ref, w2_ref, w3_ref, w4_ref, out_ref):
    dn = (((1,), (1,)), ((), ()))
    g1 = _ln_lrelu(h_ref[:, :256])
    g2 = _ln_lrelu(h_ref[:, 256:])
    w2 = w2_ref[...]
    m1 = _ln_lrelu(lax.dot_general(g1, w2, dn, preferred_element_type=jnp.float32))
    m2 = _ln_lrelu(lax.dot_general(g2, w2, dn, preferred_element_type=jnp.float32))
    w3a = w3_ref[:, :64]
    w3b = w3_ref[:, 64:]
    s = (lax.dot_general(m1, w3a, dn, preferred_element_type=jnp.float32)
         + lax.dot_general(m2, w3b, dn, preferred_element_type=jnp.float32))
    s = _ln_lrelu(s)
    out_ref[...] = lax.dot_general(s, w4_ref[...], dn,
                                   preferred_element_type=jnp.float32)


TC_B = 992


@jax.jit
def kernel(x, W1, W2, W3, W4):
    xr = x.reshape(2 * B, F)

    if TC_B > 0:
        h1_tc, h2_tc = pl.pallas_call(
            _l1_body,
            grid=(NK,),
            in_specs=[
                pl.BlockSpec((TC_B, BK), lambda k: (0, k)),
                pl.BlockSpec((TC_B, BK), lambda k: (0, NK + k)),
                pl.BlockSpec((256, BK), lambda k: (0, k)),
            ],
            out_specs=[
                pl.BlockSpec((TC_B, 256), lambda k: (0, 0)),
                pl.BlockSpec((TC_B, 256), lambda k: (0, 0)),
            ],
            out_shape=[
                jax.ShapeDtypeStruct((TC_B, 256), jnp.float32),
                jax.ShapeDtypeStruct((TC_B, 256), jnp.float32),
            ],
        )(x, x, W1)
        hcat_tc = jnp.concatenate([h1_tc, h2_tc], axis=1)

    nsc_rows = 2 * (B - TC_B)
    if nsc_rows > 0:
        h_sc = _sc_l1(xr, _w1t(W1), nsc_rows)
        hcat_sc = h_sc.reshape(B - TC_B, 512)

    if TC_B == 0:
        hcat = hcat_sc
    elif nsc_rows == 0:
        hcat = hcat_tc
    else:
        hcat = jnp.concatenate([hcat_tc, hcat_sc], axis=0)

    out = pl.pallas_call(
        _mlp_body,
        out_shape=jax.ShapeDtypeStruct((B, 1), jnp.float32),
    )(hcat, W2, W3, W4)
    return out


# speedup vs baseline: 14.5909x; 3.4550x over previous
import functools

import jax
import jax.numpy as jnp
from jax import lax
from jax.experimental import pallas as pl
from jax.experimental.pallas import tpu as pltpu

F = 40960
B = 1024
BK = 2048
NK = F // BK


def _l1_body(x1_ref, x2_ref, w1_ref, h1_ref, h2_ref):
    k = pl.program_id(0)
    wb = w1_ref[...].astype(jnp.bfloat16)
    x1b = x1_ref[...].astype(jnp.bfloat16)
    x2b = x2_ref[...].astype(jnp.bfloat16)
    dn = (((1,), (1,)), ((), ()))
    a1 = lax.dot_general(x1b, wb, dn, preferred_element_type=jnp.float32)
    a2 = lax.dot_general(x2b, wb, dn, preferred_element_type=jnp.float32)

    @pl.when(k == 0)
    def _():
        h1_ref[...] = a1
        h2_ref[...] = a2

    @pl.when(k > 0)
    def _():
        h1_ref[...] += a1
        h2_ref[...] += a2


def _ln_lrelu(v):
    mu = jnp.mean(v, axis=1, keepdims=True)
    var = jnp.mean((v - mu) ** 2, axis=1, keepdims=True)
    y = (v - mu) * lax.rsqrt(var)
    return jnp.maximum(0.05 * y, y)


def _mlp_body(h1_ref, h2_ref, w2_ref, w3_ref, w4_ref, out_ref):
    dn = (((1,), (1,)), ((), ()))
    g1 = _ln_lrelu(h1_ref[...])
    g2 = _ln_lrelu(h2_ref[...])
    w2 = w2_ref[...]
    m1 = _ln_lrelu(lax.dot_general(g1, w2, dn, preferred_element_type=jnp.float32))
    m2 = _ln_lrelu(lax.dot_general(g2, w2, dn, preferred_element_type=jnp.float32))
    w3a = w3_ref[:, :64]
    w3b = w3_ref[:, 64:]
    s = (lax.dot_general(m1, w3a, dn, preferred_element_type=jnp.float32)
         + lax.dot_general(m2, w3b, dn, preferred_element_type=jnp.float32))
    s = _ln_lrelu(s)
    out_ref[...] = lax.dot_general(s, w4_ref[...], dn,
                                   preferred_element_type=jnp.float32)


@jax.jit
def kernel(x, W1, W2, W3, W4):
    h1, h2 = pl.pallas_call(
        _l1_body,
        grid=(NK,),
        in_specs=[
            pl.BlockSpec((B, BK), lambda k: (0, k)),
            pl.BlockSpec((B, BK), lambda k: (0, NK + k)),
            pl.BlockSpec((256, BK), lambda k: (0, k)),
        ],
        out_specs=[
            pl.BlockSpec((B, 256), lambda k: (0, 0)),
            pl.BlockSpec((B, 256), lambda k: (0, 0)),
        ],
        out_shape=[
            jax.ShapeDtypeStruct((B, 256), jnp.float32),
            jax.ShapeDtypeStruct((B, 256), jnp.float32),
        ],
    )(x, x, W1)

    out = pl.pallas_call(
        _mlp_body,
        out_shape=jax.ShapeDtypeStruct((B, 1), jnp.float32),
    )(h1, h2, W2, W3, W4)
    return out
